# baseline (speedup 1.0000x reference)
import jax
import jax.numpy as jnp
from jax.experimental import pallas as pl
from jax.experimental.pallas import tpu as pltpu

_BN = 256
_BK = 2048


def _assign_kernel(x_ref, c_ref, o_ref, c2_ref, buf_ref, md2_ref, ids_ref):
    i = pl.program_id(0)
    nb = pl.num_programs(0) - 1
    cur = jax.lax.rem(i, 2)
    prev = 1 - cur

    @pl.when(i == 0)
    def _():
        cc = c_ref[...]
        c2_ref[...] = jnp.sum(cc * cc, axis=1)[None, :]
        ids_ref[...] = jax.lax.broadcasted_iota(
            jnp.int32, ids_ref.shape, 1).astype(jnp.float32)

    k = c_ref.shape[0]
    big = jnp.float32(3e9)

    @pl.when(i > 0)
    def _select_prev():
        md2 = md2_ref[prev]
        mdist = jnp.sqrt(jnp.maximum(md2, 0.0))
        t0i = jax.lax.bitcast_convert_type(mdist * mdist, jnp.int32)
        th = jnp.full_like(mdist, -jnp.inf)
        for off in range(-4, 5):
            cand = jax.lax.bitcast_convert_type(t0i + off, jnp.float32)
            ok = jnp.sqrt(jnp.maximum(cand, 0.0)) <= mdist
            th = jnp.where(ok, cand, th)
        th = jnp.where(mdist == 0.0, jnp.zeros_like(th), th)
        ids = ids_ref[...]
        q = None
        for j in range(0, k, _BK):
            d2 = buf_ref[prev, :, j:j + _BK]
            qj = jnp.min(jnp.where(d2 <= th, ids, big), axis=1) \
                + jnp.float32(j)
            q = qj if q is None else jnp.minimum(q, qj)
        o_ref[0, 0, :] = q.astype(jnp.int32)

    @pl.when(i < nb)
    def _compute_cur():
        x = x_ref[...]
        xm2 = x * -2.0
        f2 = jnp.sum(x * x, axis=1, keepdims=True)
        run = None
        for j in range(0, k, _BK):
            dotm2 = jax.lax.dot_general(
                xm2, c_ref[j:j + _BK, :], (((1,), (1,)), ((), ())),
                preferred_element_type=jnp.float32)
            d2 = (f2 + c2_ref[:, j:j + _BK]) + dotm2
            buf_ref[cur, :, j:j + _BK] = d2
            mj = jnp.min(d2, axis=1, keepdims=True)
            run = mj if run is None else jnp.minimum(run, mj)
        md2_ref[cur] = run


def kernel(inp, centroids):
    b, t, c = inp.shape
    k = centroids.shape[0]
    n = b * t
    nb = n // _BN
    x = inp.reshape(n, c)
    out = pl.pallas_call(
        _assign_kernel,
        grid=(nb + 1,),
        in_specs=[
            pl.BlockSpec((_BN, c), lambda i: (jnp.minimum(i, nb - 1), 0)),
            pl.BlockSpec((k, c), lambda i: (0, 0)),
        ],
        out_specs=pl.BlockSpec((1, 1, _BN), lambda i: (jnp.maximum(i - 1, 0), 0, 0)),
        out_shape=jax.ShapeDtypeStruct((nb, 1, _BN), jnp.int32),
        scratch_shapes=[
            pltpu.VMEM((1, k), jnp.float32),
            pltpu.VMEM((2, _BN, k), jnp.float32),
            pltpu.VMEM((2, _BN, 1), jnp.float32),
            pltpu.VMEM((_BN, _BK), jnp.float32),
        ],
    )(x, centroids)
    return out.reshape(b, t)


# speedup vs baseline: 1.2545x; 1.2545x over previous
import jax
import jax.numpy as jnp
from jax.experimental import pallas as pl
from jax.experimental.pallas import tpu as pltpu

_BN = 512
_BK = 2048


def _assign_kernel(x_ref, c_ref, o_ref, c2_ref, buf_ref, md2_ref):
    i = pl.program_id(0)
    nb = pl.num_programs(0) - 1
    cur = jax.lax.rem(i, 2)
    prev = 1 - cur

    @pl.when(i == 0)
    def _():
        cc = c_ref[...]
        c2_ref[...] = jnp.sum(cc * cc, axis=1)[None, :]

    k = c_ref.shape[0]
    bn = x_ref.shape[0]
    ids = jax.lax.broadcasted_iota(jnp.int32, (bn, _BK), 1).astype(jnp.float32)
    big = jnp.float32(3e9)

    @pl.when(i > 0)
    def _select_prev():
        md2 = md2_ref[prev]
        mdist = jnp.sqrt(jnp.maximum(md2, 0.0))
        t0i = jax.lax.bitcast_convert_type(mdist * mdist, jnp.int32)
        th = jnp.full_like(mdist, -jnp.inf)
        for off in range(-4, 5):
            cand = jax.lax.bitcast_convert_type(t0i + off, jnp.float32)
            ok = jnp.sqrt(jnp.maximum(cand, 0.0)) <= mdist
            th = jnp.where(ok, cand, th)
        th = jnp.where(mdist == 0.0, jnp.zeros_like(th), th)
        q = None
        for j in range(0, k, _BK):
            d2 = buf_ref[prev, :, j:j + _BK]
            qj = jnp.min(jnp.where(d2 <= th, ids, big), axis=1) \
                + jnp.float32(j)
            q = qj if q is None else jnp.minimum(q, qj)
        o_ref[0, 0, :] = q.astype(jnp.int32)

    @pl.when(i < nb)
    def _compute_cur():
        x = x_ref[...]
        xm2 = x * -2.0
        f2 = jnp.sum(x * x, axis=1, keepdims=True)
        run = None
        for j in range(0, k, _BK):
            dotm2 = jax.lax.dot_general(
                xm2, c_ref[j:j + _BK, :], (((1,), (1,)), ((), ())),
                preferred_element_type=jnp.float32)
            d2 = (f2 + c2_ref[:, j:j + _BK]) + dotm2
            buf_ref[cur, :, j:j + _BK] = d2
            mj = jnp.min(d2, axis=1, keepdims=True)
            run = mj if run is None else jnp.minimum(run, mj)
        md2_ref[cur] = run


def kernel(inp, centroids):
    b, t, c = inp.shape
    k = centroids.shape[0]
    n = b * t
    nb = n // _BN
    x = inp.reshape(n, c)
    out = pl.pallas_call(
        _assign_kernel,
        grid=(nb + 1,),
        in_specs=[
            pl.BlockSpec((_BN, c), lambda i: (jnp.minimum(i, nb - 1), 0)),
            pl.BlockSpec((k, c), lambda i: (0, 0)),
        ],
        out_specs=pl.BlockSpec((1, 1, _BN), lambda i: (jnp.maximum(i - 1, 0), 0, 0)),
        out_shape=jax.ShapeDtypeStruct((nb, 1, _BN), jnp.int32),
        scratch_shapes=[
            pltpu.VMEM((1, k), jnp.float32),
            pltpu.VMEM((2, _BN, k), jnp.float32),
            pltpu.VMEM((2, _BN, 1), jnp.float32),
        ],
    )(x, centroids)
    return out.reshape(b, t)
